# baseline (speedup 1.0000x reference)
import functools

import jax
import jax.numpy as jnp
from jax import lax
from jax.experimental import pallas as pl
from jax.experimental.pallas import tpu as pltpu
from jax.experimental.pallas import tpu_sc as plsc

B, S, DIM = 4, 8192, 1024
NC, NS = 2, 16
NW = NC * NS
S_PER_W = S // NW
CH = 16
NCHUNK = S_PER_W // CH
NLANE = 16

_mesh = plsc.VectorSubcoreMesh(
    core_axis_name="c", subcore_axis_name="s", num_cores=NC, num_subcores=NS
)


@functools.partial(
    pl.kernel,
    out_type=jax.ShapeDtypeStruct((B, S, DIM), jnp.float32),
    mesh=_mesh,
    scratch_types=[
        pltpu.VMEM((B, CH, DIM), jnp.float32),
        pltpu.VMEM((CH, DIM), jnp.float32),
    ],
)
def _sc_add(x_hbm, emb_hbm, out_hbm, xbuf, ebuf):
    wid = lax.axis_index("s") * NC + lax.axis_index("c")
    base = wid * S_PER_W

    def chunk_body(c, _):
        s0 = base + c * CH
        pltpu.sync_copy(emb_hbm.at[pl.ds(s0, CH)], ebuf)
        for b in range(B):
            pltpu.sync_copy(x_hbm.at[b, pl.ds(s0, CH)], xbuf.at[b])

        def row_body(r, _):
            for j in range(DIM // NLANE):
                sl = pl.ds(j * NLANE, NLANE)
                e = ebuf[r, sl]
                for b in range(B):
                    xbuf[b, r, sl] = xbuf[b, r, sl] + e
            return 0

        lax.fori_loop(0, CH, row_body, 0)
        for b in range(B):
            pltpu.sync_copy(xbuf.at[b], out_hbm.at[b, pl.ds(s0, CH)])
        return 0

    lax.fori_loop(0, NCHUNK, chunk_body, 0)


def kernel(x, embedding):
    return _sc_add(x, embedding[:S])


# speedup vs baseline: 3.1550x; 3.1550x over previous
import jax
import jax.numpy as jnp
from jax.experimental import pallas as pl

B, S, DIM = 4, 8192, 1024
BS = 512


def _add_kernel(x_ref, emb_ref, out_ref):
    out_ref[...] = x_ref[...] + emb_ref[...][None]


def kernel(x, embedding):
    emb = embedding[:S]
    grid = (S // BS,)
    return pl.pallas_call(
        _add_kernel,
        grid=grid,
        in_specs=[
            pl.BlockSpec((B, BS, DIM), lambda s: (0, s, 0)),
            pl.BlockSpec((BS, DIM), lambda s: (s, 0)),
        ],
        out_specs=pl.BlockSpec((B, BS, DIM), lambda s: (0, s, 0)),
        out_shape=jax.ShapeDtypeStruct((B, S, DIM), x.dtype),
    )(x, emb)
